# baseline (speedup 1.0000x reference)
import functools

import jax
import jax.numpy as jnp
from jax import lax
from jax.experimental import pallas as pl
from jax.experimental.pallas import tpu as pltpu
from jax.experimental.pallas import tpu_sc as plsc

B = 16
P = 512
D = 512
T = 2048
L = 16
FT = 1024


def _sc_intervals(dur):
    mesh = plsc.VectorSubcoreMesh(core_axis_name="c", subcore_axis_name="s")

    @functools.partial(
        pl.kernel,
        mesh=mesh,
        compiler_params=pltpu.CompilerParams(needs_layout_passes=False),
        out_type=jax.ShapeDtypeStruct((B, 2 * P), jnp.int32),
        scratch_types=[
            pltpu.VMEM((P,), jnp.int32),
            pltpu.VMEM((2 * P,), jnp.int32),
        ],
    )
    def body(dur_hbm, out_hbm, dur_v, se_v):
        c = lax.axis_index("c")
        s = lax.axis_index("s")
        wid = s * 2 + c

        @pl.when(wid < B)
        def _():
            b = wid
            pltpu.sync_copy(dur_hbm.at[b], dur_v)
            carry = jnp.int32(0)
            for k in range(P // L):
                v = dur_v[pl.ds(k * L, L)]
                ends = plsc.cumsum(v) + carry
                carry = carry + jnp.sum(v)
                se_v[pl.ds(k * L, L)] = ends - v
                se_v[pl.ds(P + k * L, L)] = ends
            pltpu.sync_copy(se_v, out_hbm.at[b])

    return body(dur)


def _tc_body(tot_ref, se_ref, enc_ref, out_ref, ebf_ref):
    b = pl.program_id(0)
    f = pl.program_id(1)
    start_f = f * FT
    tot = tot_ref[b]

    @pl.when(f == 0)
    def _():
        ebf_ref[...] = enc_ref[0].astype(jnp.bfloat16)

    @pl.when(start_f < tot)
    def _():
        st = se_ref[0, 0, :P]
        en = se_ref[0, 0, P:]
        t = lax.broadcasted_iota(jnp.int32, (FT, P), 0) + start_f
        oh = ((t >= st[None, :]) & (t < en[None, :])).astype(jnp.bfloat16)
        out_ref[0] = jnp.dot(oh, ebf_ref[...],
                             preferred_element_type=jnp.float32)

    @pl.when(start_f >= tot)
    def _():
        out_ref[0] = jnp.zeros((FT, D), jnp.float32)


def _tc_expand(totals, se3, enc):
    return pl.pallas_call(
        _tc_body,
        grid=(B, T // FT),
        in_specs=[
            pl.BlockSpec(memory_space=pltpu.SMEM),
            pl.BlockSpec((1, 1, 2 * P), lambda b, f: (b, 0, 0)),
            pl.BlockSpec((1, P, D), lambda b, f: (b, 0, 0)),
        ],
        out_specs=pl.BlockSpec((1, FT, D), lambda b, f: (b, f, 0)),
        out_shape=jax.ShapeDtypeStruct((B, T, D), jnp.float32),
        scratch_shapes=[pltpu.VMEM((P, D), jnp.bfloat16)],
    )(totals, se3, enc)


def kernel(encoder_output, log_durations):
    mask = (log_durations > 0).astype(jnp.int32)
    dur = (jnp.power(2.0, log_durations) + 0.0001).astype(jnp.int32) * mask
    dur = dur.reshape(B, P)
    se = _sc_intervals(dur)
    se3 = se.reshape(B, 1, 2 * P)
    totals = jnp.sum(dur, axis=1)
    return _tc_expand(totals, se3, encoder_output)
